# baseline (speedup 1.0000x reference)
import jax
import jax.numpy as jnp
from jax.experimental import pallas as pl
from jax.experimental.pallas import tpu as pltpu

_D_MODEL = 768
_D_STATE = 16
_D_CONV = 4
_D_INNER = 1536
_DT_RANK = 48
_T = 256
_TSUB = 64


_LOG2E = 1.4426950408889634
_LN2 = 0.6931471805599453


def _softplus(x):
    e = jnp.exp2(jnp.abs(x) * jnp.float32(-_LOG2E))
    return (jnp.maximum(x, 0.0)
            + jnp.log2(1.0 + e) * jnp.float32(_LN2))


def _sigmoid(x):
    return 0.5 * jnp.tanh(0.5 * x) + 0.5


def _mamba_kernel(x_ref, winT_ref, convwT_ref, convb_ref, wx_ref, wdtT_ref,
                  bdt_ref, alogT_ref, dvec_ref, woutT_ref, out_ref,
                  h_ref, tail_ref, da_ref, dbu_ref, hist_ref):
    c = pl.program_id(1)

    @pl.when(c == 0)
    def _init():
        h_ref[...] = jnp.zeros_like(h_ref)
        tail_ref[...] = jnp.zeros_like(tail_ref)

    xb = x_ref[0]
    xz = jnp.dot(xb, winT_ref[...],
                 preferred_element_type=jnp.float32)
    u = xz[:, :_D_INNER]
    z = xz[:, _D_INNER:]

    ext = jnp.concatenate([tail_ref[0:_D_CONV - 1, :], u], axis=0)
    cw = convwT_ref[...]
    acc = ext[0:_T] * cw[0:1]
    for k in range(1, _D_CONV):
        acc += ext[k:k + _T] * cw[k:k + 1]
    tail_ref[0:_D_CONV - 1, :] = u[_T - (_D_CONV - 1):_T, :]
    ucb = acc + convb_ref[...]
    uc = ucb * _sigmoid(ucb)

    xp = jnp.dot(uc.astype(jnp.bfloat16), wx_ref[...],
                 preferred_element_type=jnp.float32)
    dt_pre = jnp.dot(xp[:, 0:128].astype(jnp.bfloat16), wdtT_ref[...],
                     preferred_element_type=jnp.float32) + bdt_ref[...]
    dt = _softplus(dt_pre)
    Bm = xp[:, 128:128 + _D_STATE]
    Cm = xp[:, 256:256 + _D_STATE]

    A_l2 = -jnp.exp(alogT_ref[...]) * jnp.float32(1.4426950408889634)
    da_ref[...] = jnp.exp2(
        dt[:, None, :] * A_l2[None, :, :]).astype(jnp.bfloat16)
    dtu = (dt * uc).astype(jnp.bfloat16)
    dbu_ref[...] = dtu[:, None, :] * Bm.astype(jnp.bfloat16)[:, :, None]

    h = h_ref[...]
    for t in range(_T):
        h = da_ref[t] * h + dbu_ref[t]
        hist_ref[t] = h
    h_ref[...] = h

    Cmb = Cm.astype(jnp.bfloat16)
    j = jax.lax.broadcasted_iota(jnp.int32, (_TSUB, _TSUB * _D_STATE), 1)
    ti = jax.lax.broadcasted_iota(jnp.int32, (_TSUB, _TSUB * _D_STATE), 0)
    blkdiag = j // _D_STATE == ti
    ys_parts = []
    for s in range(0, _T, _TSUB):
        hist2 = hist_ref[s:s + _TSUB].reshape(_TSUB * _D_STATE, _D_INNER)
        Ctile = jnp.tile(Cmb[s:s + _TSUB], (1, _TSUB))
        M = jnp.where(blkdiag, Ctile, jnp.bfloat16(0.0))
        ys_parts.append(jnp.dot(M, hist2,
                                preferred_element_type=jnp.float32))
    ys = jnp.concatenate(ys_parts, axis=0)
    y = ys + uc * dvec_ref[...]
    y = y * (z * _sigmoid(z))
    out_ref[0] = jnp.dot(y.astype(jnp.bfloat16), woutT_ref[...],
                         preferred_element_type=jnp.float32)


def _run(x, W_inT, conv_wT, conv_b2, Wx_pad, W_dtT_pad, b_dt2, A_logT, D2,
         W_outT):
    B, L, dm = x.shape
    nchunk = L // _T
    full = lambda shape: pl.BlockSpec(shape, lambda b, c: (0, 0))
    return pl.pallas_call(
        _mamba_kernel,
        grid=(B, nchunk),
        in_specs=[
            pl.BlockSpec((1, _T, dm), lambda b, c: (b, c, 0)),
            full((dm, 2 * _D_INNER)),
            full((_D_CONV, _D_INNER)),
            full((1, _D_INNER)),
            full((_D_INNER, 384)),
            full((128, _D_INNER)),
            full((1, _D_INNER)),
            full((_D_STATE, _D_INNER)),
            full((1, _D_INNER)),
            full((_D_INNER, dm)),
        ],
        out_specs=pl.BlockSpec((1, _T, dm), lambda b, c: (b, c, 0)),
        out_shape=jax.ShapeDtypeStruct((B, L, dm), jnp.float32),
        scratch_shapes=[
            pltpu.VMEM((_D_STATE, _D_INNER), jnp.bfloat16),
            pltpu.VMEM((8, _D_INNER), jnp.float32),
            pltpu.VMEM((_T, _D_STATE, _D_INNER), jnp.bfloat16),
            pltpu.VMEM((_T, _D_STATE, _D_INNER), jnp.bfloat16),
            pltpu.VMEM((_T, _D_STATE, _D_INNER), jnp.bfloat16),
        ],
        compiler_params=pltpu.CompilerParams(
            dimension_semantics=("parallel", "arbitrary"),
            vmem_limit_bytes=56 * 1024 * 1024,
        ),
        name="mamba_fused",
    )(x, W_inT, conv_wT, conv_b2, Wx_pad, W_dtT_pad, b_dt2, A_logT, D2, W_outT)


def kernel(x, W_in, conv_w, conv_b, W_xproj, W_dt, b_dt, A_log, D, W_out):
    B = x.shape[0]
    xbf = x.astype(jnp.bfloat16)
    W_inT = W_in.T.astype(jnp.bfloat16)
    conv_wT = conv_w.T
    conv_b2 = conv_b[None, :]
    WxT = W_xproj.T
    Wx_pad = (jnp.zeros((_D_INNER, 384), x.dtype)
              .at[:, 0:_DT_RANK].set(WxT[:, :_DT_RANK])
              .at[:, 128:128 + _D_STATE].set(WxT[:, _DT_RANK:_DT_RANK + _D_STATE])
              .at[:, 256:256 + _D_STATE].set(WxT[:, _DT_RANK + _D_STATE:])
              ).astype(jnp.bfloat16)
    W_dtT_pad = (jnp.zeros((128, _D_INNER), x.dtype)
                 .at[:_DT_RANK, :].set(W_dt.T)).astype(jnp.bfloat16)
    b_dt2 = b_dt[None, :]
    A_logT = A_log.T
    D2 = D[None, :]
    W_outT = W_out.T.astype(jnp.bfloat16)
    return _run(xbf, W_inT, conv_wT, conv_b2, Wx_pad, W_dtT_pad, b_dt2,
                A_logT, D2, W_outT)


# speedup vs baseline: 1.0395x; 1.0395x over previous
import jax
import jax.numpy as jnp
from jax.experimental import pallas as pl
from jax.experimental.pallas import tpu as pltpu

_D_MODEL = 768
_D_STATE = 16
_D_CONV = 4
_D_INNER = 1536
_DT_RANK = 48
_T = 256
_TSUB = 64


_LOG2E = 1.4426950408889634
_LN2 = 0.6931471805599453


def _softplus(x):
    e = jnp.exp2(jnp.abs(x) * jnp.float32(-_LOG2E))
    return (jnp.maximum(x, 0.0)
            + jnp.log2(1.0 + e) * jnp.float32(_LN2))


def _sigmoid(x):
    return 0.5 * jnp.tanh(0.5 * x) + 0.5


def _mamba_kernel(x_ref, winT_ref, convwT_ref, convb_ref, wx_ref, wdtT_ref,
                  bdt_ref, alogT_ref, dvec_ref, woutT_ref, out_ref,
                  h_ref, tail_ref, da_ref, dbu_ref, hist_ref):
    c = pl.program_id(1)

    @pl.when(c == 0)
    def _init():
        h_ref[...] = jnp.zeros_like(h_ref)
        tail_ref[...] = jnp.zeros_like(tail_ref)

    xb = x_ref[0].astype(jnp.bfloat16)
    xz = jnp.dot(xb, winT_ref[...],
                 preferred_element_type=jnp.float32)
    u = xz[:, :_D_INNER]
    z = xz[:, _D_INNER:]

    ext = jnp.concatenate([tail_ref[0:_D_CONV - 1, :], u], axis=0)
    cw = convwT_ref[...]
    acc = ext[0:_T] * cw[0:1]
    for k in range(1, _D_CONV):
        acc += ext[k:k + _T] * cw[k:k + 1]
    tail_ref[0:_D_CONV - 1, :] = u[_T - (_D_CONV - 1):_T, :]
    ucb = acc + convb_ref[...]
    uc = ucb * _sigmoid(ucb)

    xp = jnp.dot(uc.astype(jnp.bfloat16), wx_ref[...],
                 preferred_element_type=jnp.float32)
    dt_pre = jnp.dot(xp[:, 0:128].astype(jnp.bfloat16), wdtT_ref[...],
                     preferred_element_type=jnp.float32) + bdt_ref[...]
    dt = _softplus(dt_pre)
    Bm = xp[:, 128:128 + _D_STATE]
    Cm = xp[:, 256:256 + _D_STATE]

    A_l2 = -jnp.exp(alogT_ref[...]) * jnp.float32(1.4426950408889634)
    da_ref[...] = jnp.exp2(
        dt[:, None, :] * A_l2[None, :, :]).astype(jnp.bfloat16)
    dtu = (dt * uc).astype(jnp.bfloat16)
    dbu_ref[...] = dtu[:, None, :] * Bm.astype(jnp.bfloat16)[:, :, None]

    h = h_ref[...]
    for t in range(_T):
        h = da_ref[t] * h + dbu_ref[t]
        hist_ref[t] = h
    h_ref[...] = h

    Cmb = Cm.astype(jnp.bfloat16)
    j = jax.lax.broadcasted_iota(jnp.int32, (_TSUB, _TSUB * _D_STATE), 1)
    ti = jax.lax.broadcasted_iota(jnp.int32, (_TSUB, _TSUB * _D_STATE), 0)
    blkdiag = j // _D_STATE == ti
    ys_parts = []
    for s in range(0, _T, _TSUB):
        hist2 = hist_ref[s:s + _TSUB].reshape(_TSUB * _D_STATE, _D_INNER)
        Ctile = jnp.tile(Cmb[s:s + _TSUB], (1, _TSUB))
        M = jnp.where(blkdiag, Ctile, jnp.bfloat16(0.0))
        ys_parts.append(jnp.dot(M, hist2,
                                preferred_element_type=jnp.float32))
    ys = jnp.concatenate(ys_parts, axis=0)
    y = ys + uc * dvec_ref[...]
    y = y * (z * _sigmoid(z))
    out_ref[0] = jnp.dot(y.astype(jnp.bfloat16), woutT_ref[...],
                         preferred_element_type=jnp.float32)


def _run(x, W_inT, conv_wT, conv_b2, Wx_pad, W_dtT_pad, b_dt2, A_logT, D2,
         W_outT):
    B, L, dm = x.shape
    nchunk = L // _T
    full = lambda shape: pl.BlockSpec(shape, lambda b, c: (0, 0))
    return pl.pallas_call(
        _mamba_kernel,
        grid=(B, nchunk),
        in_specs=[
            pl.BlockSpec((1, _T, dm), lambda b, c: (b, c, 0)),
            full((dm, 2 * _D_INNER)),
            full((_D_CONV, _D_INNER)),
            full((1, _D_INNER)),
            full((_D_INNER, 384)),
            full((128, _D_INNER)),
            full((1, _D_INNER)),
            full((_D_STATE, _D_INNER)),
            full((1, _D_INNER)),
            full((_D_INNER, dm)),
        ],
        out_specs=pl.BlockSpec((1, _T, dm), lambda b, c: (b, c, 0)),
        out_shape=jax.ShapeDtypeStruct((B, L, dm), jnp.float32),
        scratch_shapes=[
            pltpu.VMEM((_D_STATE, _D_INNER), jnp.bfloat16),
            pltpu.VMEM((8, _D_INNER), jnp.float32),
            pltpu.VMEM((_T, _D_STATE, _D_INNER), jnp.bfloat16),
            pltpu.VMEM((_T, _D_STATE, _D_INNER), jnp.bfloat16),
            pltpu.VMEM((_T, _D_STATE, _D_INNER), jnp.bfloat16),
        ],
        compiler_params=pltpu.CompilerParams(
            dimension_semantics=("parallel", "arbitrary"),
            vmem_limit_bytes=56 * 1024 * 1024,
        ),
        name="mamba_fused",
    )(x, W_inT, conv_wT, conv_b2, Wx_pad, W_dtT_pad, b_dt2, A_logT, D2, W_outT)


def kernel(x, W_in, conv_w, conv_b, W_xproj, W_dt, b_dt, A_log, D, W_out):
    B = x.shape[0]
    W_inT = W_in.T.astype(jnp.bfloat16)
    conv_wT = conv_w.T
    conv_b2 = conv_b[None, :]
    WxT = W_xproj.T
    Wx_pad = (jnp.zeros((_D_INNER, 384), x.dtype)
              .at[:, 0:_DT_RANK].set(WxT[:, :_DT_RANK])
              .at[:, 128:128 + _D_STATE].set(WxT[:, _DT_RANK:_DT_RANK + _D_STATE])
              .at[:, 256:256 + _D_STATE].set(WxT[:, _DT_RANK + _D_STATE:])
              ).astype(jnp.bfloat16)
    W_dtT_pad = (jnp.zeros((128, _D_INNER), x.dtype)
                 .at[:_DT_RANK, :].set(W_dt.T)).astype(jnp.bfloat16)
    b_dt2 = b_dt[None, :]
    A_logT = A_log.T
    D2 = D[None, :]
    W_outT = W_out.T.astype(jnp.bfloat16)
    return _run(x, W_inT, conv_wT, conv_b2, Wx_pad, W_dtT_pad, b_dt2,
                A_logT, D2, W_outT)
